# baseline (speedup 1.0000x reference)
import functools

import jax
import jax.numpy as jnp
from jax.experimental import pallas as pl

K = 1024
D = 64
BETA = 0.25
BB = 1


def _vq_kernel(x_ref, emb_ref, q_ref, idx_ref, loss_ref):
    b = pl.program_id(0)

    x = jnp.concatenate([x_ref[i] for i in range(BB)], axis=1)
    emb = emb_ref[...]

    e2 = jnp.sum(emb * emb, axis=1, keepdims=True)
    x2 = jnp.sum(x * x, axis=0, keepdims=True)

    mm2 = jax.lax.dot_general(
        emb * (-2.0), x,
        dimension_numbers=(((1,), (0,)), ((), ())),
        preferred_element_type=jnp.float32,
    )
    s = (x2 + e2) + mm2

    m = jnp.min(s, axis=0, keepdims=True)
    mask = jnp.where(s == m, 1.0, 0.0).astype(jnp.bfloat16)

    kvec = jax.lax.broadcasted_iota(jnp.int32, (K, 1), 0).astype(jnp.float32)
    ones = jnp.ones((K, 1), jnp.float32)
    g = jnp.concatenate([emb, kvec, ones], axis=1)
    g_hi = g.astype(jnp.bfloat16)
    g_lo = (g - g_hi.astype(jnp.float32)).astype(jnp.bfloat16)

    dims = (((0,), (0,)), ((), ()))
    agg = jax.lax.dot_general(
        g_hi, mask, dimension_numbers=dims,
        preferred_element_type=jnp.float32,
    ) + jax.lax.dot_general(
        g_lo, mask, dimension_numbers=dims,
        preferred_element_type=jnp.float32,
    )

    cnt = agg[D + 1:D + 2]
    rec = 1.0 / cnt
    q = agg[:D] * rec
    idx = jnp.floor(agg[D:D + 1] * rec + 0.5).astype(jnp.int32)

    HW = x.shape[1] // BB
    for i in range(BB):
        q_ref[i] = q[:, i * HW:(i + 1) * HW]
        idx_ref[i, 0, :] = idx[0, i * HW:(i + 1) * HW]

    part = jnp.sum(m).reshape(1, 1)

    @pl.when(b == 0)
    def _init():
        loss_ref[...] = part

    @pl.when(b != 0)
    def _acc():
        loss_ref[...] += part


@functools.partial(jax.jit, static_argnames=("interpret",))
def kernel(enc_pred, embeddings, interpret=False):
    B, d, H, W = enc_pred.shape
    HW = H * W
    x = enc_pred.reshape(B, d, HW)

    q, idx, loss_sum = pl.pallas_call(
        _vq_kernel,
        grid=(B // BB,),
        in_specs=[
            pl.BlockSpec((BB, d, HW), lambda b: (b, 0, 0)),
            pl.BlockSpec((K, D), lambda b: (0, 0)),
        ],
        out_specs=[
            pl.BlockSpec((BB, d, HW), lambda b: (b, 0, 0)),
            pl.BlockSpec((BB, 1, HW), lambda b: (b, 0, 0)),
            pl.BlockSpec((1, 1), lambda b: (0, 0)),
        ],
        out_shape=[
            jax.ShapeDtypeStruct((B, d, HW), jnp.float32),
            jax.ShapeDtypeStruct((B, 1, HW), jnp.int32),
            jax.ShapeDtypeStruct((1, 1), jnp.float32),
        ],
        interpret=interpret,
    )(x, embeddings)

    quantized_out = q.reshape(B, d, H, W)
    indices_out = idx.reshape(B, 1, H, W)
    loss = BETA * (loss_sum[0, 0] / jnp.float32(B * HW * D))
    return (quantized_out, loss, indices_out)


# speedup vs baseline: 1.2893x; 1.2893x over previous
import functools

import jax
import jax.numpy as jnp
from jax.experimental import pallas as pl

K = 1024
D = 64
BETA = 0.25
BB = 2


def _vq_kernel(x_ref, emb_ref, q_ref, idx_ref, loss_ref):
    b = pl.program_id(0)

    x = jnp.concatenate([x_ref[i] for i in range(BB)], axis=1)
    emb = emb_ref[...]

    e2 = jnp.sum(emb * emb, axis=1, keepdims=True)
    x2 = jnp.sum(x * x, axis=0, keepdims=True)

    mm2 = jax.lax.dot_general(
        emb * (-2.0), x,
        dimension_numbers=(((1,), (0,)), ((), ())),
        preferred_element_type=jnp.float32,
    )
    s = (x2 + e2) + mm2

    m = jnp.min(s, axis=0, keepdims=True)
    mask = jnp.where(s == m, 1.0, 0.0)

    kvec = jax.lax.broadcasted_iota(jnp.int32, (K, 1), 0).astype(jnp.float32)
    ones = jnp.ones((K, 1), jnp.float32)
    g = jnp.concatenate([emb, kvec, ones], axis=1)

    agg = jax.lax.dot_general(
        g, mask,
        dimension_numbers=(((0,), (0,)), ((), ())),
        preferred_element_type=jnp.float32,
    )

    cnt = agg[D + 1:D + 2]
    rec = 1.0 / cnt
    q = agg[:D] * rec
    idx = jnp.floor(agg[D:D + 1] * rec + 0.5).astype(jnp.int32)

    HW = x.shape[1] // BB
    for i in range(BB):
        q_ref[i] = q[:, i * HW:(i + 1) * HW]
        idx_ref[i, 0, :] = idx[0, i * HW:(i + 1) * HW]

    part = jnp.sum(m).reshape(1, 1)

    @pl.when(b == 0)
    def _init():
        loss_ref[...] = part

    @pl.when(b != 0)
    def _acc():
        loss_ref[...] += part


@functools.partial(jax.jit, static_argnames=("interpret",))
def kernel(enc_pred, embeddings, interpret=False):
    B, d, H, W = enc_pred.shape
    HW = H * W
    x = enc_pred.reshape(B, d, HW)

    q, idx, loss_sum = pl.pallas_call(
        _vq_kernel,
        grid=(B // BB,),
        in_specs=[
            pl.BlockSpec((BB, d, HW), lambda b: (b, 0, 0)),
            pl.BlockSpec((K, D), lambda b: (0, 0)),
        ],
        out_specs=[
            pl.BlockSpec((BB, d, HW), lambda b: (b, 0, 0)),
            pl.BlockSpec((BB, 1, HW), lambda b: (b, 0, 0)),
            pl.BlockSpec((1, 1), lambda b: (0, 0)),
        ],
        out_shape=[
            jax.ShapeDtypeStruct((B, d, HW), jnp.float32),
            jax.ShapeDtypeStruct((B, 1, HW), jnp.int32),
            jax.ShapeDtypeStruct((1, 1), jnp.float32),
        ],
        interpret=interpret,
    )(x, embeddings)

    quantized_out = q.reshape(B, d, H, W)
    indices_out = idx.reshape(B, 1, H, W)
    loss = BETA * (loss_sum[0, 0] / jnp.float32(B * HW * D))
    return (quantized_out, loss, indices_out)


# speedup vs baseline: 1.3282x; 1.0302x over previous
import functools

import jax
import jax.numpy as jnp
from jax.experimental import pallas as pl

K = 1024
D = 64
BETA = 0.25
BB = 4


def _vq_kernel(x_ref, emb_ref, q_ref, idx_ref, loss_ref):
    b = pl.program_id(0)

    x = jnp.concatenate([x_ref[i] for i in range(BB)], axis=1)
    emb = emb_ref[...]

    e2 = jnp.sum(emb * emb, axis=1, keepdims=True)
    x2 = jnp.sum(x * x, axis=0, keepdims=True)

    mm2 = jax.lax.dot_general(
        emb * (-2.0), x,
        dimension_numbers=(((1,), (0,)), ((), ())),
        preferred_element_type=jnp.float32,
    )
    s = (x2 + e2) + mm2

    m = jnp.min(s, axis=0, keepdims=True)
    mask = jnp.where(s == m, 1.0, 0.0)

    kvec = jax.lax.broadcasted_iota(jnp.int32, (K, 1), 0).astype(jnp.float32)
    ones = jnp.ones((K, 1), jnp.float32)
    g = jnp.concatenate([emb, kvec, ones], axis=1)

    agg = jax.lax.dot_general(
        g, mask,
        dimension_numbers=(((0,), (0,)), ((), ())),
        preferred_element_type=jnp.float32,
    )

    cnt = agg[D + 1:D + 2]
    rec = 1.0 / cnt
    q = agg[:D] * rec
    idx = jnp.floor(agg[D:D + 1] * rec + 0.5).astype(jnp.int32)

    HW = x.shape[1] // BB
    for i in range(BB):
        q_ref[i] = q[:, i * HW:(i + 1) * HW]
        idx_ref[i, 0, :] = idx[0, i * HW:(i + 1) * HW]

    part = jnp.sum(m).reshape(1, 1)

    @pl.when(b == 0)
    def _init():
        loss_ref[...] = part

    @pl.when(b != 0)
    def _acc():
        loss_ref[...] += part


@functools.partial(jax.jit, static_argnames=("interpret",))
def kernel(enc_pred, embeddings, interpret=False):
    B, d, H, W = enc_pred.shape
    HW = H * W
    x = enc_pred.reshape(B, d, HW)

    q, idx, loss_sum = pl.pallas_call(
        _vq_kernel,
        grid=(B // BB,),
        in_specs=[
            pl.BlockSpec((BB, d, HW), lambda b: (b, 0, 0)),
            pl.BlockSpec((K, D), lambda b: (0, 0)),
        ],
        out_specs=[
            pl.BlockSpec((BB, d, HW), lambda b: (b, 0, 0)),
            pl.BlockSpec((BB, 1, HW), lambda b: (b, 0, 0)),
            pl.BlockSpec((1, 1), lambda b: (0, 0)),
        ],
        out_shape=[
            jax.ShapeDtypeStruct((B, d, HW), jnp.float32),
            jax.ShapeDtypeStruct((B, 1, HW), jnp.int32),
            jax.ShapeDtypeStruct((1, 1), jnp.float32),
        ],
        interpret=interpret,
    )(x, embeddings)

    quantized_out = q.reshape(B, d, H, W)
    indices_out = idx.reshape(B, 1, H, W)
    loss = BETA * (loss_sum[0, 0] / jnp.float32(B * HW * D))
    return (quantized_out, loss, indices_out)
